# baseline (speedup 1.0000x reference)
import functools

import jax
import jax.numpy as jnp
from jax import lax
from jax.experimental import pallas as pl
from jax.experimental.pallas import tpu as pltpu
from jax.experimental.pallas import tpu_sc as plsc

N = 10000
E = 320000
BB = 40
RPT = 632
RPT_LAST = N - 15 * RPT

_f32 = jnp.float32


def _k1_body(x_ref, w_ref, p64_ref, ts_ref, td_ref, init_ref, m_ref):
    y = jnp.dot(x_ref[...], w_ref[...], preferred_element_type=_f32)
    h1 = y[:, :64]
    a_s = y[:, 64:72]
    a_d = y[:, 72:80]
    m = jnp.maximum(jnp.max(a_s, axis=0) + jnp.max(a_d, axis=0), 0.0)
    p64 = p64_ref[...]
    v = a_s + a_d
    ex = jnp.exp(jnp.maximum(v, 0.2 * v) - m[None, :])
    exx = jnp.dot(ex, p64, preferred_element_type=_f32)
    adx = jnp.dot(a_d, p64, preferred_element_type=_f32)
    init = jnp.concatenate([h1 * exx, exx], axis=1)
    ts_ref[...] = jnp.concatenate(
        [h1, jnp.dot(a_s, p64, preferred_element_type=_f32)], axis=1)
    td_ref[...] = jnp.concatenate([adx, adx], axis=1)
    init_ref[...] = jnp.concatenate([init, jnp.zeros_like(init)], axis=0)
    mx = jnp.dot(m.reshape(1, 8), p64, preferred_element_type=_f32)
    m_ref[...] = jnp.concatenate([mx, mx], axis=1)


def _k3a_body(acc_ref, b1_ref, w2_ref, t1_ref, p128_ref,
              h2_ref, aa_ref, m8_ref, m_ref):
    a = acc_ref[...]
    acc = a[:N] + a[N:]
    num = acc[:, :64]
    den = jnp.dot(acc[:, 64:128], t1_ref[...], preferred_element_type=_f32)
    o1 = num / (den + 1e-16) + b1_ref[...]
    h = jnp.where(o1 > 0, o1, jnp.exp(jnp.minimum(o1, 0.0)) - 1.0)
    y = jnp.dot(h, w2_ref[...], preferred_element_type=_f32)
    a_s = y[:, 128:136]
    a_d = y[:, 136:144]
    m = jnp.maximum(jnp.max(a_s, axis=0) + jnp.max(a_d, axis=0), 0.0)
    h2_ref[...] = y[:, :128]
    aa_ref[...] = jnp.concatenate([a_s, a_d], axis=1)
    m8_ref[...] = m.reshape(1, 8)
    m_ref[...] = jnp.dot(m.reshape(1, 8), p128_ref[...],
                         preferred_element_type=_f32)


_K3B_R = 2000


def _k3b_body(h2_ref, aa_ref, m8_ref, p128_ref, ts_ref, td_ref, init_ref):
    half_hi = pl.program_id(0) >= 5
    h2 = h2_ref[...]
    a_s = aa_ref[:, :8]
    a_d = aa_ref[:, 8:]
    p128 = p128_ref[...]
    v = a_s + a_d
    ex = jnp.exp(jnp.maximum(v, 0.2 * v) - m8_ref[...])
    exx = jnp.dot(ex, p128, preferred_element_type=_f32)
    asx = jnp.dot(a_s, p128, preferred_element_type=_f32)
    adx = jnp.dot(a_d, p128, preferred_element_type=_f32)
    hexx = h2 * exx

    def pick(z):
        return jnp.where(half_hi, z[:, 64:], z[:, :64])
    ts_ref[...] = jnp.concatenate([pick(h2), pick(asx)], axis=1)
    td_ref[...] = jnp.concatenate([pick(adx), pick(adx)], axis=1)
    init_ref[...] = jnp.concatenate([pick(hexx), pick(exx)], axis=1)


def _k5_body(acc_ref, b2_ref, q_ref, o_ref):
    a = acc_ref[...]
    lo = a[:N]
    hi = a[N:]
    o = jnp.concatenate([lo[:, :64] / (lo[:, 64:] + 1e-16),
                         hi[:, :64] / (hi[:, 64:] + 1e-16)], axis=1)
    o = jnp.dot(o, q_ref[...], preferred_element_type=_f32) * 0.125 + b2_ref[...]
    mx = jnp.max(o, axis=1, keepdims=True)
    e = o - mx
    lse = jnp.log(jnp.sum(jnp.exp(e), axis=1, keepdims=True))
    o_ref[...] = e - lse


def _tc_call(body, out_shapes, *args):
    return pl.pallas_call(
        body,
        out_shape=[jax.ShapeDtypeStruct(s, _f32) for s in out_shapes],
    )(*args)


def _edge_body(split, ts_hbm, td_hbm, eint_hbm, init_hbm,
               m_hbm, out_hbm,
               iraw0, iraw1, igs0, igd0, isc0, igs1, igd1, isc1,
               sb0, db0, mg0, sb1, db1, mg1, mbuf, acc,
               si0, si1, sg0, sg1, ss0, ss1):
    cid = lax.axis_index("c")
    sid = lax.axis_index("s")
    r0 = sid * RPT
    if split:
        ept = E // 16
        tbl_off = cid * N
        cb = sid * (ept // BB)
    else:
        ept = E // 32
        tbl_off = None
        cb = (cid * 16 + sid) * (ept // BB)
    nchunk = ept // BB

    @pl.when(sid != 15)
    def _():
        pltpu.sync_copy(init_hbm.at[pl.ds(cid * N + r0, RPT)],
                        acc.at[pl.ds(r0, RPT)])

    @pl.when(sid == 15)
    def _():
        pltpu.sync_copy(init_hbm.at[pl.ds(cid * N + r0, RPT_LAST)],
                        acc.at[pl.ds(r0, RPT_LAST)])

    plsc.subcore_barrier()

    pltpu.sync_copy(m_hbm.at[pl.ds(cid * 64, 64)], mbuf)
    mvec = [mbuf[pl.ds(16 * j, 16)] for j in range(4)]

    def idx_slice(i):
        return eint_hbm.at[pl.ds((cb + i) * (2 * BB), 2 * BB)]

    def build(iraw, igs, igd, isc):
        for g in range(BB // 16):
            so = pl.ds(16 * g, 16)
            s = iraw[so]
            d = iraw[pl.ds(BB + 16 * g, 16)]
            isc[so] = d
            if tbl_off is None:
                igs[so] = s
                igd[so] = d
            else:
                igs[so] = s + tbl_off
                igd[so] = d + tbl_off

    def start_gathers(igs, igd, sb, db, sg):
        pltpu.async_copy(ts_hbm.at[igs], sb, sg)
        pltpu.async_copy(td_hbm.at[igd], db, sg)

    def wait_gathers(igs, igd, sb, db, sg):
        pltpu.make_async_copy(ts_hbm.at[igs], sb, sg).wait()
        pltpu.make_async_copy(td_hbm.at[igd], db, sg).wait()

    def compute(sb, db, mg):
        def quad(p, c):
            for u in range(4):
                b = 4 * p + u
                for j in range(4):
                    v = sb[b, pl.ds(64 + 16 * j, 16)] + db[b, pl.ds(16 * j, 16)]
                    e = jnp.exp(jnp.maximum(v, 0.2 * v) - mvec[j])
                    mg[b, pl.ds(64 + 16 * j, 16)] = e
                    mg[b, pl.ds(16 * j, 16)] = sb[b, pl.ds(16 * j, 16)] * e
            return c
        lax.fori_loop(0, BB // 4, quad, 0)

    buf0 = (iraw0, igs0, igd0, isc0, sb0, db0, mg0, si0, sg0, ss0)
    buf1 = (iraw1, igs1, igd1, isc1, sb1, db1, mg1, si1, sg1, ss1)

    def body(i, cur, nxt):
        iraw_c, igs_c, igd_c, isc_c, sb_c, db_c, mg_c, si_c, sg_c, ss_c = cur
        iraw_n, igs_n, igd_n, isc_n, sb_n, db_n, mg_n, si_n, sg_n, ss_n = nxt

        @pl.when(i >= 2)
        def _():
            pltpu.make_async_copy(mg_c, acc.at[isc_c], ss_c).wait()

        @pl.when(i + 1 < nchunk)
        def _():
            pltpu.make_async_copy(idx_slice(i + 1), iraw_n, si_n).wait()
            build(iraw_n, igs_n, igd_n, isc_n)

            @pl.when(i + 2 < nchunk)
            def _():
                pltpu.async_copy(idx_slice(i + 2), iraw_c, si_c)
            start_gathers(igs_n, igd_n, sb_n, db_n, sg_n)
        wait_gathers(igs_c, igd_c, sb_c, db_c, sg_c)
        compute(sb_c, db_c, mg_c)
        pltpu.async_copy(mg_c, acc.at[isc_c], ss_c, add=True)

    pltpu.sync_copy(idx_slice(0), iraw0)
    build(iraw0, igs0, igd0, isc0)
    pltpu.async_copy(idx_slice(1), iraw1, si1)
    start_gathers(igs0, igd0, sb0, db0, sg0)

    def pairs(p, c):
        body(2 * p, buf0, buf1)
        body(2 * p + 1, buf1, buf0)
        return c
    lax.fori_loop(0, nchunk // 2, pairs, 0)
    if nchunk % 2 == 1:
        body(jnp.int32(nchunk - 1), buf0, buf1)

    pltpu.make_async_copy(mg0, acc.at[isc0], ss0).wait()
    pltpu.make_async_copy(mg1, acc.at[isc1], ss1).wait()
    plsc.subcore_barrier()

    @pl.when(sid != 15)
    def _():
        pltpu.sync_copy(acc.at[pl.ds(r0, RPT)],
                        out_hbm.at[pl.ds(cid * N + r0, RPT)])

    @pl.when(sid == 15)
    def _():
        pltpu.sync_copy(acc.at[pl.ds(r0, RPT_LAST)],
                        out_hbm.at[pl.ds(cid * N + r0, RPT_LAST)])


def _edge_layer(split, ts, td, eint, init, m):
    mesh = plsc.VectorSubcoreMesh(core_axis_name="c", subcore_axis_name="s")
    kern = functools.partial(
        pl.kernel,
        out_type=jax.ShapeDtypeStruct((2 * N, 128), _f32),
        mesh=mesh,
        scratch_types=[
            pltpu.VMEM((2 * BB,), jnp.int32),
            pltpu.VMEM((2 * BB,), jnp.int32),
            pltpu.VMEM((BB,), jnp.int32),
            pltpu.VMEM((BB,), jnp.int32),
            pltpu.VMEM((BB,), jnp.int32),
            pltpu.VMEM((BB,), jnp.int32),
            pltpu.VMEM((BB,), jnp.int32),
            pltpu.VMEM((BB,), jnp.int32),
            pltpu.VMEM((BB, 128), _f32),
            pltpu.VMEM((BB, 128), _f32),
            pltpu.VMEM((BB, 128), _f32),
            pltpu.VMEM((BB, 128), _f32),
            pltpu.VMEM((BB, 128), _f32),
            pltpu.VMEM((BB, 128), _f32),
            pltpu.VMEM((64,), _f32),
            pltpu.VMEM_SHARED((N, 128), _f32),
            pltpu.SemaphoreType.DMA,
            pltpu.SemaphoreType.DMA,
            pltpu.SemaphoreType.DMA,
            pltpu.SemaphoreType.DMA,
            pltpu.SemaphoreType.DMA,
            pltpu.SemaphoreType.DMA,
        ],
    )(functools.partial(_edge_body, split))
    return kern(ts, td, eint, init, m)


def kernel(x, edge_index, W1, att_src1, att_dst1, b1, W2, att_src2, att_dst2, b2):
    ws1 = jnp.einsum("dhc,hc->dh", W1.reshape(128, 8, 8), att_src1)
    wd1 = jnp.einsum("dhc,hc->dh", W1.reshape(128, 8, 8), att_dst1)
    ws2 = jnp.einsum("dhc,hc->dh", W2.reshape(64, 8, 16), att_src2)
    wd2 = jnp.einsum("dhc,hc->dh", W2.reshape(64, 8, 16), att_dst2)
    wcat1 = jnp.concatenate([W1, ws1, wd1], axis=1)
    wcat2 = jnp.concatenate([W2, ws2, wd2], axis=1)
    p64 = jnp.repeat(jnp.eye(8, dtype=_f32), 8, axis=1)
    p128 = jnp.repeat(jnp.eye(8, dtype=_f32), 16, axis=1)
    t1 = jnp.kron(jnp.eye(8, dtype=_f32), jnp.full((8, 8), 0.125, _f32))
    q = jnp.tile(jnp.eye(16, dtype=_f32), (8, 1))
    eint = jnp.concatenate(
        [edge_index[0].reshape(-1, BB), edge_index[1].reshape(-1, BB)],
        axis=1).reshape(-1)

    ts1, td1, init1, m1 = _tc_call(
        _k1_body, [(N, 128), (N, 128), (2 * N, 128), (1, 128)],
        x, wcat1, p64)
    acc1 = _edge_layer(False, ts1, td1, eint, init1, m1.reshape(128))

    h2, aa2, m8, m2 = _tc_call(
        _k3a_body, [(N, 128), (N, 16), (1, 8), (1, 128)],
        acc1, b1.reshape(1, 64), wcat2, t1, p128)
    r = _K3B_R
    ts2, td2, init2 = pl.pallas_call(
        _k3b_body,
        grid=(10,),
        in_specs=[
            pl.BlockSpec((r, 128), lambda i: (i % 5, 0)),
            pl.BlockSpec((r, 16), lambda i: (i % 5, 0)),
            pl.BlockSpec((1, 8), lambda i: (0, 0)),
            pl.BlockSpec((8, 128), lambda i: (0, 0)),
        ],
        out_specs=[
            pl.BlockSpec((r, 128), lambda i: (i, 0)),
            pl.BlockSpec((r, 128), lambda i: (i, 0)),
            pl.BlockSpec((r, 128), lambda i: (i, 0)),
        ],
        out_shape=[jax.ShapeDtypeStruct((2 * N, 128), _f32)] * 3,
    )(h2, aa2, m8, p128)
    acc2 = _edge_layer(True, ts2, td2, eint, init2, m2.reshape(128))

    (out,) = _tc_call(_k5_body, [(N, 16)], acc2, b2.reshape(1, 16), q)
    return out


# speedup vs baseline: 11.8789x; 11.8789x over previous
import functools

import jax
import jax.numpy as jnp
from jax import lax
from jax.experimental import pallas as pl
from jax.experimental.pallas import tpu as pltpu
from jax.experimental.pallas import tpu_sc as plsc

N = 10000
E = 320000
BB = 80
RPT = 632
RPT_LAST = N - 15 * RPT

_f32 = jnp.float32


def _k1_body(x_ref, w_ref, p64_ref, ts_ref, td_ref, init_ref, m_ref):
    y = jnp.dot(x_ref[...], w_ref[...], preferred_element_type=_f32)
    h1 = y[:, :64]
    a_s = y[:, 64:72]
    a_d = y[:, 72:80]
    m = jnp.maximum(jnp.max(a_s, axis=0) + jnp.max(a_d, axis=0), 0.0)
    p64 = p64_ref[...]
    v = a_s + a_d
    ex = jnp.exp(jnp.maximum(v, 0.2 * v) - m[None, :])
    exx = jnp.dot(ex, p64, preferred_element_type=_f32)
    adx = jnp.dot(a_d, p64, preferred_element_type=_f32)
    init = jnp.concatenate([h1 * exx, exx], axis=1)
    ts_ref[...] = jnp.concatenate(
        [h1, jnp.dot(a_s, p64, preferred_element_type=_f32)], axis=1)
    td_ref[...] = jnp.concatenate([adx, adx], axis=1)
    init_ref[...] = jnp.concatenate([init, jnp.zeros_like(init)], axis=0)
    mx = jnp.dot(m.reshape(1, 8), p64, preferred_element_type=_f32)
    m_ref[...] = jnp.concatenate([mx, mx], axis=1)


def _k3a_body(acc_ref, b1_ref, w2_ref, t1_ref, p128_ref,
              h2_ref, aa_ref, m8_ref, m_ref):
    a = acc_ref[...]
    acc = a[:N] + a[N:]
    num = acc[:, :64]
    den = jnp.dot(acc[:, 64:128], t1_ref[...], preferred_element_type=_f32)
    o1 = num / (den + 1e-16) + b1_ref[...]
    h = jnp.where(o1 > 0, o1, jnp.exp(jnp.minimum(o1, 0.0)) - 1.0)
    y = jnp.dot(h, w2_ref[...], preferred_element_type=_f32)
    a_s = y[:, 128:136]
    a_d = y[:, 136:144]
    m = jnp.maximum(jnp.max(a_s, axis=0) + jnp.max(a_d, axis=0), 0.0)
    h2_ref[...] = y[:, :128]
    aa_ref[...] = jnp.concatenate([a_s, a_d], axis=1)
    m8_ref[...] = m.reshape(1, 8)
    m_ref[...] = jnp.dot(m.reshape(1, 8), p128_ref[...],
                         preferred_element_type=_f32)


_K3B_R = 2000


def _k3b_body(h2_ref, aa_ref, m8_ref, p128_ref, ts_ref, td_ref, init_ref):
    half_hi = pl.program_id(0) >= 5
    h2 = h2_ref[...]
    a_s = aa_ref[:, :8]
    a_d = aa_ref[:, 8:]
    p128 = p128_ref[...]
    v = a_s + a_d
    ex = jnp.exp(jnp.maximum(v, 0.2 * v) - m8_ref[...])
    exx = jnp.dot(ex, p128, preferred_element_type=_f32)
    asx = jnp.dot(a_s, p128, preferred_element_type=_f32)
    adx = jnp.dot(a_d, p128, preferred_element_type=_f32)
    hexx = h2 * exx

    def pick(z):
        return jnp.where(half_hi, z[:, 64:], z[:, :64])
    ts_ref[...] = jnp.concatenate([pick(h2), pick(asx)], axis=1)
    td_ref[...] = jnp.concatenate([pick(adx), pick(adx)], axis=1)
    init_ref[...] = jnp.concatenate([pick(hexx), pick(exx)], axis=1)


def _k5_body(acc_ref, b2_ref, q_ref, o_ref):
    a = acc_ref[...]
    lo = a[:N]
    hi = a[N:]
    o = jnp.concatenate([lo[:, :64] / (lo[:, 64:] + 1e-16),
                         hi[:, :64] / (hi[:, 64:] + 1e-16)], axis=1)
    o = jnp.dot(o, q_ref[...], preferred_element_type=_f32) * 0.125 + b2_ref[...]
    mx = jnp.max(o, axis=1, keepdims=True)
    e = o - mx
    lse = jnp.log(jnp.sum(jnp.exp(e), axis=1, keepdims=True))
    o_ref[...] = e - lse


def _tc_call(body, out_shapes, *args):
    return pl.pallas_call(
        body,
        out_shape=[jax.ShapeDtypeStruct(s, _f32) for s in out_shapes],
    )(*args)


def _edge_body(split, ts_hbm, td_hbm, eint_hbm, init_hbm,
               m_hbm, out_hbm,
               iraw0, iraw1, igs0, igd0, isc0, igs1, igd1, isc1,
               sb0, db0, sb1, db1, mbuf, acc,
               si0, si1, sg0, sg1):
    cid = lax.axis_index("c")
    sid = lax.axis_index("s")
    r0 = sid * RPT
    if split:
        ept = E // 16
        tbl_off = cid * N
        cb = sid * (ept // BB)
    else:
        ept = E // 32
        tbl_off = None
        cb = (cid * 16 + sid) * (ept // BB)
    nchunk = ept // BB

    @pl.when(sid != 15)
    def _():
        pltpu.sync_copy(init_hbm.at[pl.ds(cid * N + r0, RPT)],
                        acc.at[pl.ds(r0, RPT)])

    @pl.when(sid == 15)
    def _():
        pltpu.sync_copy(init_hbm.at[pl.ds(cid * N + r0, RPT_LAST)],
                        acc.at[pl.ds(r0, RPT_LAST)])

    plsc.subcore_barrier()

    pltpu.sync_copy(m_hbm.at[pl.ds(cid * 64, 64)], mbuf)
    mvec = [mbuf[pl.ds(16 * j, 16)] for j in range(4)]

    def idx_slice(i):
        return eint_hbm.at[pl.ds((cb + i) * (2 * BB), 2 * BB)]

    def build(iraw, igs, igd, isc):
        for g in range(BB // 16):
            so = pl.ds(16 * g, 16)
            s = iraw[so]
            d = iraw[pl.ds(BB + 16 * g, 16)]
            isc[so] = d
            if tbl_off is None:
                igs[so] = s
                igd[so] = d
            else:
                igs[so] = s + tbl_off
                igd[so] = d + tbl_off

    def start_gathers(igs, igd, sb, db, sg):
        pltpu.async_copy(ts_hbm.at[igs], sb, sg)
        pltpu.async_copy(td_hbm.at[igd], db, sg)

    def wait_gathers(igs, igd, sb, db, sg):
        pltpu.make_async_copy(ts_hbm.at[igs], sb, sg).wait()
        pltpu.make_async_copy(td_hbm.at[igd], db, sg).wait()

    def compute(sb, db):
        def quad(p, c):
            for u in range(4):
                b = 4 * p + u
                for j in range(4):
                    v = sb[b, pl.ds(64 + 16 * j, 16)] + db[b, pl.ds(16 * j, 16)]
                    e = jnp.exp(jnp.maximum(v, 0.2 * v) - mvec[j])
                    sb[b, pl.ds(64 + 16 * j, 16)] = e
                    sb[b, pl.ds(16 * j, 16)] = sb[b, pl.ds(16 * j, 16)] * e
            return c
        lax.fori_loop(0, BB // 4, quad, 0)

    buf0 = (iraw0, igs0, igd0, isc0, sb0, db0, si0, sg0)
    buf1 = (iraw1, igs1, igd1, isc1, sb1, db1, si1, sg1)

    def body(i, cur, nxt):
        iraw_c, igs_c, igd_c, isc_c, sb_c, db_c, si_c, sg_c = cur
        iraw_n, igs_n, igd_n, isc_n, sb_n, db_n, si_n, sg_n = nxt

        @pl.when(i + 1 < nchunk)
        def _():
            pltpu.make_async_copy(idx_slice(i + 1), iraw_n, si_n).wait()
            build(iraw_n, igs_n, igd_n, isc_n)

            @pl.when(i + 2 < nchunk)
            def _():
                pltpu.async_copy(idx_slice(i + 2), iraw_c, si_c)
            start_gathers(igs_n, igd_n, sb_n, db_n, sg_n)
        wait_gathers(igs_c, igd_c, sb_c, db_c, sg_c)
        compute(sb_c, db_c)
        pltpu.sync_copy(sb_c, acc.at[isc_c], add=True)

    pltpu.sync_copy(idx_slice(0), iraw0)
    build(iraw0, igs0, igd0, isc0)
    pltpu.async_copy(idx_slice(1), iraw1, si1)
    start_gathers(igs0, igd0, sb0, db0, sg0)

    def pairs(p, c):
        body(2 * p, buf0, buf1)
        body(2 * p + 1, buf1, buf0)
        return c
    lax.fori_loop(0, nchunk // 2, pairs, 0)
    if nchunk % 2 == 1:
        body(jnp.int32(nchunk - 1), buf0, buf1)

    plsc.subcore_barrier()

    @pl.when(sid != 15)
    def _():
        pltpu.sync_copy(acc.at[pl.ds(r0, RPT)],
                        out_hbm.at[pl.ds(cid * N + r0, RPT)])

    @pl.when(sid == 15)
    def _():
        pltpu.sync_copy(acc.at[pl.ds(r0, RPT_LAST)],
                        out_hbm.at[pl.ds(cid * N + r0, RPT_LAST)])


def _edge_layer(split, ts, td, eint, init, m):
    mesh = plsc.VectorSubcoreMesh(core_axis_name="c", subcore_axis_name="s")
    kern = functools.partial(
        pl.kernel,
        out_type=jax.ShapeDtypeStruct((2 * N, 128), _f32),
        mesh=mesh,
        scratch_types=[
            pltpu.VMEM((2 * BB,), jnp.int32),
            pltpu.VMEM((2 * BB,), jnp.int32),
            pltpu.VMEM((BB,), jnp.int32),
            pltpu.VMEM((BB,), jnp.int32),
            pltpu.VMEM((BB,), jnp.int32),
            pltpu.VMEM((BB,), jnp.int32),
            pltpu.VMEM((BB,), jnp.int32),
            pltpu.VMEM((BB,), jnp.int32),
            pltpu.VMEM((BB, 128), _f32),
            pltpu.VMEM((BB, 128), _f32),
            pltpu.VMEM((BB, 128), _f32),
            pltpu.VMEM((BB, 128), _f32),
            pltpu.VMEM((64,), _f32),
            pltpu.VMEM_SHARED((N, 128), _f32),
            pltpu.SemaphoreType.DMA,
            pltpu.SemaphoreType.DMA,
            pltpu.SemaphoreType.DMA,
            pltpu.SemaphoreType.DMA,
        ],
    )(functools.partial(_edge_body, split))
    return kern(ts, td, eint, init, m)


def kernel(x, edge_index, W1, att_src1, att_dst1, b1, W2, att_src2, att_dst2, b2):
    ws1 = jnp.einsum("dhc,hc->dh", W1.reshape(128, 8, 8), att_src1)
    wd1 = jnp.einsum("dhc,hc->dh", W1.reshape(128, 8, 8), att_dst1)
    ws2 = jnp.einsum("dhc,hc->dh", W2.reshape(64, 8, 16), att_src2)
    wd2 = jnp.einsum("dhc,hc->dh", W2.reshape(64, 8, 16), att_dst2)
    wcat1 = jnp.concatenate([W1, ws1, wd1], axis=1)
    wcat2 = jnp.concatenate([W2, ws2, wd2], axis=1)
    p64 = jnp.repeat(jnp.eye(8, dtype=_f32), 8, axis=1)
    p128 = jnp.repeat(jnp.eye(8, dtype=_f32), 16, axis=1)
    t1 = jnp.kron(jnp.eye(8, dtype=_f32), jnp.full((8, 8), 0.125, _f32))
    q = jnp.tile(jnp.eye(16, dtype=_f32), (8, 1))
    eint = jnp.concatenate(
        [edge_index[0].reshape(-1, BB), edge_index[1].reshape(-1, BB)],
        axis=1).reshape(-1)

    ts1, td1, init1, m1 = _tc_call(
        _k1_body, [(N, 128), (N, 128), (2 * N, 128), (1, 128)],
        x, wcat1, p64)
    acc1 = _edge_layer(False, ts1, td1, eint, init1, m1.reshape(128))

    h2, aa2, m8, m2 = _tc_call(
        _k3a_body, [(N, 128), (N, 16), (1, 8), (1, 128)],
        acc1, b1.reshape(1, 64), wcat2, t1, p128)
    r = _K3B_R
    ts2, td2, init2 = pl.pallas_call(
        _k3b_body,
        grid=(10,),
        in_specs=[
            pl.BlockSpec((r, 128), lambda i: (i % 5, 0)),
            pl.BlockSpec((r, 16), lambda i: (i % 5, 0)),
            pl.BlockSpec((1, 8), lambda i: (0, 0)),
            pl.BlockSpec((8, 128), lambda i: (0, 0)),
        ],
        out_specs=[
            pl.BlockSpec((r, 128), lambda i: (i, 0)),
            pl.BlockSpec((r, 128), lambda i: (i, 0)),
            pl.BlockSpec((r, 128), lambda i: (i, 0)),
        ],
        out_shape=[jax.ShapeDtypeStruct((2 * N, 128), _f32)] * 3,
    )(h2, aa2, m8, p128)
    acc2 = _edge_layer(True, ts2, td2, eint, init2, m2.reshape(128))

    (out,) = _tc_call(_k5_body, [(N, 16)], acc2, b2.reshape(1, 16), q)
    return out
